# baseline (speedup 1.0000x reference)
import functools

import jax
import jax.numpy as jnp
from jax import lax
from jax.experimental import pallas as pl
from jax.experimental.pallas import tpu as pltpu
from jax.experimental.pallas import tpu_sc as plsc

_B = 4096
_NN = 13
_NC = 26
_V = 100000
_D = 64
_T = 1 + _NN + _NC

_NW = 32
_BPW = _B // _NW
_CB = 32
_NCHUNK = _BPW // _CB

_NCAT = _CB * _NC
_NNUM = _CB * (1 + _NN)


def _sc_body(xnum_hbm, xcat_hbm, w_hbm, bias_hbm, tab_hbm, cls_hbm, out_hbm,
             idx_v, oidx_v, nidx_v, foff_v, catpos_v, numpos_v,
             rows_v, numcls_v, xnum_v, w_v, bias_v, cls_v,
             gsem, s1sem, s2sem):
    cid = lax.axis_index("c")
    sid = lax.axis_index("s")
    wid = sid * 2 + cid
    base = wid * _BPW

    pltpu.sync_copy(w_hbm, w_v)
    pltpu.sync_copy(bias_hbm, bias_v)
    pltpu.sync_copy(cls_hbm, cls_v)

    for j in range(_NCAT // 16):
        s = pl.ds(j * 16, 16)
        p = lax.iota(jnp.int32, 16) + (j * 16)
        f = lax.rem(p, _NC)
        foff_v[s] = f * _V
        catpos_v[s] = lax.div(p, _NC) * _T + (1 + _NN) + f
    for j in range(_NNUM // 16):
        s = pl.ds(j * 16, 16)
        q = lax.iota(jnp.int32, 16) + (j * 16)
        numpos_v[s] = lax.div(q, 1 + _NN) * _T + lax.rem(q, 1 + _NN)

    scat_prev = []
    for c in range(_NCHUNK):
        b0 = base + c * _CB
        row0 = b0 * _T

        pltpu.sync_copy(xcat_hbm.at[pl.ds(b0 * _NC, _NCAT)], idx_v)
        for j in range(_NCAT // 16):
            s = pl.ds(j * 16, 16)
            idx_v[s] = idx_v[s] + foff_v[s]

        for d in scat_prev:
            d.wait()

        gat = pltpu.async_copy(tab_hbm.at[idx_v], rows_v, gsem)

        for j in range(_NCAT // 16):
            s = pl.ds(j * 16, 16)
            oidx_v[s] = catpos_v[s] + row0
        for j in range(_NNUM // 16):
            s = pl.ds(j * 16, 16)
            nidx_v[s] = numpos_v[s] + row0

        pltpu.sync_copy(xnum_hbm.at[pl.ds(b0, _CB)], xnum_v)

        def _row(b, _):
            xv = xnum_v[b, :]
            for k in range(_D // 16):
                numcls_v[b * (1 + _NN), pl.ds(k * 16, 16)] = cls_v[pl.ds(k * 16, 16)]
            for f in range(_NN):
                xs = xv[f]
                for k in range(_D // 16):
                    s = pl.ds(k * 16, 16)
                    numcls_v[b * (1 + _NN) + 1 + f, s] = xs * w_v[f, s] + bias_v[f, s]
            return 0

        lax.fori_loop(0, _CB, _row, 0)

        s1 = pltpu.async_copy(numcls_v, out_hbm.at[nidx_v], s1sem)
        gat.wait()
        s2 = pltpu.async_copy(rows_v, out_hbm.at[oidx_v], s2sem)
        scat_prev = [s1, s2]

    for d in scat_prev:
        d.wait()


@jax.jit
def _tokenize(x_num_flat, x_cat_flat, W_num, b_num, tables_flat, cls_flat):
    mesh = plsc.VectorSubcoreMesh(core_axis_name="c", subcore_axis_name="s")
    kern = pl.kernel(
        _sc_body,
        out_type=jax.ShapeDtypeStruct((_B * _T, _D), jnp.float32),
        mesh=mesh,
        scratch_types=[
            pltpu.VMEM((_NCAT,), jnp.int32),
            pltpu.VMEM((_NCAT,), jnp.int32),
            pltpu.VMEM((_NNUM,), jnp.int32),
            pltpu.VMEM((_NCAT,), jnp.int32),
            pltpu.VMEM((_NCAT,), jnp.int32),
            pltpu.VMEM((_NNUM,), jnp.int32),
            pltpu.VMEM((_NCAT, _D), jnp.float32),
            pltpu.VMEM((_NNUM, _D), jnp.float32),
            pltpu.VMEM((_CB, 16), jnp.float32),
            pltpu.VMEM((_NN, _D), jnp.float32),
            pltpu.VMEM((_NN, _D), jnp.float32),
            pltpu.VMEM((_D,), jnp.float32),
            pltpu.SemaphoreType.DMA,
            pltpu.SemaphoreType.DMA,
            pltpu.SemaphoreType.DMA,
        ],
        compiler_params=pltpu.CompilerParams(use_tc_tiling_on_sc=False),
    )
    return kern(x_num_flat, x_cat_flat, W_num, b_num, tables_flat, cls_flat)


def kernel(x_num, x_cat, W_num, b_num, tables, cls_token):
    x_num_pad = jnp.pad(x_num, ((0, 0), (0, 16 - _NN)))
    x_cat_flat = x_cat.astype(jnp.int32).reshape(_B * _NC)
    tables_flat = tables.reshape(_NC * _V, _D)
    cls_flat = cls_token.reshape(_D)
    out = _tokenize(x_num_pad, x_cat_flat, W_num, b_num, tables_flat, cls_flat)
    return out.reshape(_B, _T, _D)


# speedup vs baseline: 1.0019x; 1.0019x over previous
import jax
import jax.numpy as jnp
from jax import lax
from jax.experimental import pallas as pl
from jax.experimental.pallas import tpu as pltpu
from jax.experimental.pallas import tpu_sc as plsc

_B = 4096
_NN = 13
_NC = 26
_V = 100000
_D = 64
_T = 1 + _NN + _NC

_NW = 32
_BPW = _B // _NW
_CB = 16
_NCHUNK = _BPW // _CB

_NCAT = _CB * _NC
_NNUM = _CB * (1 + _NN)


def _sc_body(xnum_hbm, xcat_hbm, w_hbm, bias_hbm, tab_hbm, cls_hbm, out_hbm,
             qidx_v, h_v, oidx_v, nidx_v, fq_v, catpos_v, numpos_v,
             rows_v, numcls_v, xnum_v, w_v, bias_v, cls_v,
             gsem, s1sem, s2sem):
    cid = lax.axis_index("c")
    sid = lax.axis_index("s")
    wid = sid * 2 + cid
    base = wid * _BPW

    pltpu.sync_copy(w_hbm, w_v)
    pltpu.sync_copy(bias_hbm, bias_v)
    pltpu.sync_copy(cls_hbm, cls_v)

    for j in range(_NCAT // 16):
        s = pl.ds(j * 16, 16)
        p = lax.iota(jnp.int32, 16) + (j * 16)
        f = lax.rem(p, _NC)
        fq_v[s] = f * (_V // 2)
        catpos_v[s] = lax.div(p, _NC) * _T + (1 + _NN) + f
    for j in range(_NNUM // 16):
        s = pl.ds(j * 16, 16)
        q = lax.iota(jnp.int32, 16) + (j * 16)
        numpos_v[s] = lax.div(q, 1 + _NN) * _T + lax.rem(q, 1 + _NN)

    scat_prev = []
    for c in range(_NCHUNK):
        b0 = base + c * _CB
        row0 = b0 * _T

        pltpu.sync_copy(xcat_hbm.at[pl.ds(b0 * _NC, _NCAT)], qidx_v)
        for j in range(_NCAT // 16):
            s = pl.ds(j * 16, 16)
            v = qidx_v[s]
            h_v[s] = jnp.bitwise_and(v, 1)
            qidx_v[s] = jnp.right_shift(v, 1) + fq_v[s]

        for d in scat_prev:
            d.wait()

        gat = pltpu.async_copy(tab_hbm.at[qidx_v], rows_v, gsem)

        for j in range(_NCAT // 16):
            s = pl.ds(j * 16, 16)
            oidx_v[s] = catpos_v[s] + row0
        for j in range(_NNUM // 16):
            s = pl.ds(j * 16, 16)
            nidx_v[s] = numpos_v[s] + row0

        pltpu.sync_copy(xnum_hbm.at[pl.ds(b0 * 16, _CB * 16)], xnum_v)

        def _row(b, _):
            xv = xnum_v[pl.ds(b * 16, 16)]
            for k in range(_D // 16):
                numcls_v[b * (1 + _NN), pl.ds(k * 16, 16)] = cls_v[pl.ds(k * 16, 16)]
            for f in range(_NN):
                xs = xv[f]
                for k in range(_D // 16):
                    s = pl.ds(k * 16, 16)
                    numcls_v[b * (1 + _NN) + 1 + f, s] = xs * w_v[pl.ds(f * _D + k * 16, 16)] + bias_v[pl.ds(f * _D + k * 16, 16)]
            return 0

        lax.fori_loop(0, _CB, _row, 0)

        s1 = pltpu.async_copy(numcls_v, out_hbm.at[nidx_v], s1sem)

        gat.wait()

        def _grp(g, _):
            hv = h_v[pl.ds(g * 16, 16)]
            for i in range(16):
                j = g * 16 + i
                off = hv[i] * _D
                for k in range(_D // 16):
                    t = rows_v[j, pl.ds(off + k * 16, 16)]
                    rows_v[j, pl.ds(k * 16, 16)] = t
            return 0

        lax.fori_loop(0, _NCAT // 16, _grp, 0)

        s2 = pltpu.async_copy(rows_v, out_hbm.at[oidx_v], s2sem)
        scat_prev = [s1, s2]

    for d in scat_prev:
        d.wait()


@jax.jit
def _tokenize(x_num_flat, x_cat_flat, w_flat, bias_flat, tables_pairs, cls_flat):
    mesh = plsc.VectorSubcoreMesh(core_axis_name="c", subcore_axis_name="s")
    kern = pl.kernel(
        _sc_body,
        out_type=jax.ShapeDtypeStruct((_B * _T, 2 * _D), jnp.float32),
        mesh=mesh,
        scratch_types=[
            pltpu.VMEM((_NCAT,), jnp.int32),
            pltpu.VMEM((_NCAT,), jnp.int32),
            pltpu.VMEM((_NCAT,), jnp.int32),
            pltpu.VMEM((_NNUM,), jnp.int32),
            pltpu.VMEM((_NCAT,), jnp.int32),
            pltpu.VMEM((_NCAT,), jnp.int32),
            pltpu.VMEM((_NNUM,), jnp.int32),
            pltpu.VMEM((_NCAT, 2 * _D), jnp.float32),
            pltpu.VMEM((_NNUM, 2 * _D), jnp.float32),
            pltpu.VMEM((_CB * 16,), jnp.float32),
            pltpu.VMEM((_NN * _D,), jnp.float32),
            pltpu.VMEM((_NN * _D,), jnp.float32),
            pltpu.VMEM((_D,), jnp.float32),
            pltpu.SemaphoreType.DMA,
            pltpu.SemaphoreType.DMA,
            pltpu.SemaphoreType.DMA,
        ],
        compiler_params=pltpu.CompilerParams(use_tc_tiling_on_sc=True),
    )
    return kern(x_num_flat, x_cat_flat, w_flat, bias_flat, tables_pairs, cls_flat)


def kernel(x_num, x_cat, W_num, b_num, tables, cls_token):
    x_num_flat = jnp.pad(x_num, ((0, 0), (0, 16 - _NN))).reshape(_B * 16)
    x_cat_flat = x_cat.astype(jnp.int32).reshape(_B * _NC)
    tables_pairs = tables.reshape(_NC * _V // 2, 2 * _D)
    w_flat = W_num.reshape(_NN * _D)
    bias_flat = b_num.reshape(_NN * _D)
    cls_flat = cls_token.reshape(_D)
    out2 = _tokenize(x_num_flat, x_cat_flat, w_flat, bias_flat,
                     tables_pairs, cls_flat)
    return out2[:, :_D].reshape(_B, _T, _D)


# speedup vs baseline: 2.4389x; 2.4343x over previous
import jax
import jax.numpy as jnp
from jax import lax
from jax.experimental import pallas as pl
from jax.experimental.pallas import tpu as pltpu
from jax.experimental.pallas import tpu_sc as plsc

_B = 4096
_NN = 13
_NC = 26
_V = 100000
_D = 64
_T = 1 + _NN + _NC

_NW = 32
_BPW = _B // _NW
_CB = 8
_NCHUNK = _BPW // _CB

_NCAT = _CB * _NC


def _sc_body(xnum_hbm, xcat_hbm, w_hbm, bias_hbm, tab_hbm, cls_hbm, out_hbm,
             idx_v, asm_v, xnum_v, w_v, bias_v, cls_v,
             gsem, osem):
    cid = lax.axis_index("c")
    sid = lax.axis_index("s")
    wid = sid * 2 + cid
    base = wid * _BPW

    pltpu.sync_copy(w_hbm, w_v)
    pltpu.sync_copy(bias_hbm, bias_v)
    pltpu.sync_copy(cls_hbm, cls_v)

    def _chunk(c, _):
        b0 = base + c * _CB

        pltpu.sync_copy(xcat_hbm.at[pl.ds(b0 * _NC, _NCAT)], idx_v)
        pltpu.sync_copy(xnum_hbm.at[pl.ds(b0 * 16, _CB * 16)], xnum_v)

        descs = []
        for j in range(_NCAT // 16):
            vv = idx_v[pl.ds(j * 16, 16)]
            for i in range(16):
                p = j * 16 + i
                b, f = divmod(p, _NC)
                d = pltpu.async_copy(
                    tab_hbm.at[vv[i] + f * _V], asm_v.at[b, 1 + _NN + f], gsem)
                descs.append(d)

        for b in range(_CB):
            xv = xnum_v[pl.ds(b * 16, 16)]
            for k in range(_D // 16):
                asm_v[b, 0, pl.ds(k * 16, 16)] = cls_v[pl.ds(k * 16, 16)]
            for f in range(_NN):
                xs = xv[f]
                for k in range(_D // 16):
                    s = pl.ds(k * 16, 16)
                    asm_v[b, 1 + f, s] = (
                        xs * w_v[pl.ds(f * _D + k * 16, 16)]
                        + bias_v[pl.ds(f * _D + k * 16, 16)])

        for d in descs:
            d.wait()

        pltpu.sync_copy(asm_v, out_hbm.at[pl.ds(b0, _CB)])
        return 0

    lax.fori_loop(0, _NCHUNK, _chunk, 0)


@jax.jit
def _tokenize(x_num_flat, x_cat_flat, w_flat, bias_flat, tables, cls_flat):
    mesh = plsc.VectorSubcoreMesh(core_axis_name="c", subcore_axis_name="s")
    kern = pl.kernel(
        _sc_body,
        out_type=jax.ShapeDtypeStruct((_B, _T, _D), jnp.float32),
        mesh=mesh,
        scratch_types=[
            pltpu.VMEM((_NCAT,), jnp.int32),
            pltpu.VMEM((_CB, _T, _D), jnp.float32),
            pltpu.VMEM((_CB * 16,), jnp.float32),
            pltpu.VMEM((_NN * _D,), jnp.float32),
            pltpu.VMEM((_NN * _D,), jnp.float32),
            pltpu.VMEM((_D,), jnp.float32),
            pltpu.SemaphoreType.DMA,
            pltpu.SemaphoreType.DMA,
        ],
        compiler_params=pltpu.CompilerParams(use_tc_tiling_on_sc=True),
    )
    return kern(x_num_flat, x_cat_flat, w_flat, bias_flat, tables, cls_flat)


def kernel(x_num, x_cat, W_num, b_num, tables, cls_token):
    x_num_flat = jnp.pad(x_num, ((0, 0), (0, 16 - _NN))).reshape(_B * 16)
    x_cat_flat = x_cat.astype(jnp.int32).reshape(_B * _NC)
    tables_flat = tables.reshape(_NC * _V, _D)
    w_flat = W_num.reshape(_NN * _D)
    bias_flat = b_num.reshape(_NN * _D)
    cls_flat = cls_token.reshape(_D)
    return _tokenize(x_num_flat, x_cat_flat, w_flat, bias_flat,
                     tables_flat, cls_flat)


# speedup vs baseline: 2.6548x; 1.0885x over previous
import jax
import jax.numpy as jnp
from jax import lax
from jax.experimental import pallas as pl
from jax.experimental.pallas import tpu as pltpu
from jax.experimental.pallas import tpu_sc as plsc

_B = 4096
_NN = 13
_NC = 26
_V = 100000
_D = 64
_T = 1 + _NN + _NC

_NW = 32
_BPW = _B // _NW
_CB = 8
_NCHUNK = _BPW // _CB

_NCAT = _CB * _NC


def _sc_body(xnum_hbm, xcat_hbm, w_hbm, bias_hbm, tab_hbm, cls_hbm, out_hbm,
             idx_v, asm0_v, asm1_v, xnum_v, w_v, bias_v, cls_v,
             gsem, osem0, osem1):
    cid = lax.axis_index("c")
    sid = lax.axis_index("s")
    wid = sid * 2 + cid
    base = wid * _BPW

    pltpu.sync_copy(w_hbm, w_v)
    pltpu.sync_copy(bias_hbm, bias_v)
    pltpu.sync_copy(cls_hbm, cls_v)
    pltpu.sync_copy(xcat_hbm.at[pl.ds(base * _NC, _BPW * _NC)], idx_v)
    pltpu.sync_copy(xnum_hbm.at[pl.ds(base * 16, _BPW * 16)], xnum_v)

    def _chunk(c, asm_v, osem, wait_prev):
        b0 = base + c * _CB
        dst = out_hbm.at[pl.ds(b0 * _T, _CB * _T)]

        @pl.when(wait_prev)
        def _():
            pltpu.make_async_copy(asm_v, dst, osem).wait()

        o = c * _NCAT
        descs = []
        for j in range(_NCAT // 16):
            vv = idx_v[pl.ds(o + j * 16, 16)]
            for i in range(16):
                p = j * 16 + i
                b, f = divmod(p, _NC)
                d = pltpu.async_copy(
                    tab_hbm.at[vv[i] + f * _V],
                    asm_v.at[b * _T + 1 + _NN + f], gsem)
                descs.append(d)

        for b in range(_CB):
            xv = xnum_v[pl.ds((c * _CB + b) * 16, 16)]
            for k in range(_D // 16):
                asm_v[b * _T, pl.ds(k * 16, 16)] = cls_v[pl.ds(k * 16, 16)]
            for f in range(_NN):
                xs = xv[f]
                for k in range(_D // 16):
                    s = pl.ds(k * 16, 16)
                    asm_v[b * _T + 1 + f, s] = (
                        xs * w_v[pl.ds(f * _D + k * 16, 16)]
                        + bias_v[pl.ds(f * _D + k * 16, 16)])

        for d in descs:
            d.wait()

        pltpu.async_copy(asm_v, dst, osem)

    def _pair(p, _):
        _chunk(2 * p, asm0_v, osem0, p >= 1)
        _chunk(2 * p + 1, asm1_v, osem1, p >= 1)
        return 0

    lax.fori_loop(0, _NCHUNK // 2, _pair, 0)

    last = out_hbm.at[pl.ds(base * _T, _CB * _T)]
    pltpu.make_async_copy(asm0_v, last, osem0).wait()
    pltpu.make_async_copy(asm1_v, last, osem1).wait()


@jax.jit
def _tokenize(x_num_flat, x_cat_flat, w_flat, bias_flat, tables_flat, cls_flat):
    mesh = plsc.VectorSubcoreMesh(core_axis_name="c", subcore_axis_name="s")
    kern = pl.kernel(
        _sc_body,
        out_type=jax.ShapeDtypeStruct((_B * _T, _D), jnp.float32),
        mesh=mesh,
        scratch_types=[
            pltpu.VMEM((_BPW * _NC,), jnp.int32),
            pltpu.VMEM((_CB * _T, _D), jnp.float32),
            pltpu.VMEM((_CB * _T, _D), jnp.float32),
            pltpu.VMEM((_BPW * 16,), jnp.float32),
            pltpu.VMEM((_NN * _D,), jnp.float32),
            pltpu.VMEM((_NN * _D,), jnp.float32),
            pltpu.VMEM((_D,), jnp.float32),
            pltpu.SemaphoreType.DMA,
            pltpu.SemaphoreType.DMA,
            pltpu.SemaphoreType.DMA,
        ],
        compiler_params=pltpu.CompilerParams(use_tc_tiling_on_sc=True),
    )
    return kern(x_num_flat, x_cat_flat, w_flat, bias_flat, tables_flat, cls_flat)


def kernel(x_num, x_cat, W_num, b_num, tables, cls_token):
    x_num_flat = jnp.pad(x_num, ((0, 0), (0, 16 - _NN))).reshape(_B * 16)
    x_cat_flat = x_cat.astype(jnp.int32).reshape(_B * _NC)
    tables_flat = tables.reshape(_NC * _V, _D)
    w_flat = W_num.reshape(_NN * _D)
    bias_flat = b_num.reshape(_NN * _D)
    cls_flat = cls_token.reshape(_D)
    out2 = _tokenize(x_num_flat, x_cat_flat, w_flat, bias_flat,
                     tables_flat, cls_flat)
    return out2.reshape(_B, _T, _D)


# speedup vs baseline: 2.6635x; 1.0033x over previous
import jax
import jax.numpy as jnp
from jax import lax
from jax.experimental import pallas as pl
from jax.experimental.pallas import tpu as pltpu
from jax.experimental.pallas import tpu_sc as plsc

_B = 4096
_NN = 13
_NC = 26
_V = 100000
_D = 64
_T = 1 + _NN + _NC

_NW = 32
_BPW = _B // _NW
_CB = 8
_NCHUNK = _BPW // _CB

_NCAT = _CB * _NC


def _sc_body(xnum_hbm, xcat_hbm, w_hbm, bias_hbm, tab_hbm, cls_hbm, out_hbm,
             idx_v, asm0_v, asm1_v, xnum_v, w_v, bias_v, cls_v,
             gsem0, gsem1, osem0, osem1):
    cid = lax.axis_index("c")
    sid = lax.axis_index("s")
    wid = sid * 2 + cid
    base = wid * _BPW

    pltpu.sync_copy(w_hbm, w_v)
    pltpu.sync_copy(bias_hbm, bias_v)
    pltpu.sync_copy(cls_hbm, cls_v)
    pltpu.sync_copy(xcat_hbm.at[pl.ds(base * _NC, _BPW * _NC)], idx_v)
    pltpu.sync_copy(xnum_hbm.at[pl.ds(base * 16, _BPW * 16)], xnum_v)

    def _fire(c, asm_v, gsem, osem, wait_prev):
        b0 = base + c * _CB
        dst = out_hbm.at[pl.ds(b0 * _T, _CB * _T)]

        @pl.when(wait_prev)
        def _():
            pltpu.make_async_copy(asm_v, dst, osem).wait()

        o = c * _NCAT
        descs = []
        for j in range(_NCAT // 16):
            vv = idx_v[pl.ds(o + j * 16, 16)]
            for i in range(16):
                p = j * 16 + i
                b, f = divmod(p, _NC)
                d = pltpu.async_copy(
                    tab_hbm.at[vv[i] + f * _V],
                    asm_v.at[b * _T + 1 + _NN + f], gsem)
                descs.append(d)
        return dst, descs

    def _finish(c, asm_v, osem, dst, descs):
        for b in range(_CB):
            xv = xnum_v[pl.ds((c * _CB + b) * 16, 16)]
            for k in range(_D // 16):
                asm_v[b * _T, pl.ds(k * 16, 16)] = cls_v[pl.ds(k * 16, 16)]
            for f in range(_NN):
                xs = xv[f]
                for k in range(_D // 16):
                    s = pl.ds(k * 16, 16)
                    asm_v[b * _T + 1 + f, s] = (
                        xs * w_v[pl.ds(f * _D + k * 16, 16)]
                        + bias_v[pl.ds(f * _D + k * 16, 16)])

        for d in descs:
            d.wait()

        pltpu.async_copy(asm_v, dst, osem)

    def _pair(p, _):
        dst0, descs0 = _fire(2 * p, asm0_v, gsem0, osem0, p >= 1)
        dst1, descs1 = _fire(2 * p + 1, asm1_v, gsem1, osem1, p >= 1)
        _finish(2 * p, asm0_v, osem0, dst0, descs0)
        _finish(2 * p + 1, asm1_v, osem1, dst1, descs1)
        return 0

    lax.fori_loop(0, _NCHUNK // 2, _pair, 0)

    last = out_hbm.at[pl.ds(base * _T, _CB * _T)]
    pltpu.make_async_copy(asm0_v, last, osem0).wait()
    pltpu.make_async_copy(asm1_v, last, osem1).wait()


@jax.jit
def _tokenize(x_num_flat, x_cat_flat, w_flat, bias_flat, tables_flat, cls_flat):
    mesh = plsc.VectorSubcoreMesh(core_axis_name="c", subcore_axis_name="s")
    kern = pl.kernel(
        _sc_body,
        out_type=jax.ShapeDtypeStruct((_B * _T, _D), jnp.float32),
        mesh=mesh,
        scratch_types=[
            pltpu.VMEM((_BPW * _NC,), jnp.int32),
            pltpu.VMEM((_CB * _T, _D), jnp.float32),
            pltpu.VMEM((_CB * _T, _D), jnp.float32),
            pltpu.VMEM((_BPW * 16,), jnp.float32),
            pltpu.VMEM((_NN * _D,), jnp.float32),
            pltpu.VMEM((_NN * _D,), jnp.float32),
            pltpu.VMEM((_D,), jnp.float32),
            pltpu.SemaphoreType.DMA,
            pltpu.SemaphoreType.DMA,
            pltpu.SemaphoreType.DMA,
            pltpu.SemaphoreType.DMA,
        ],
        compiler_params=pltpu.CompilerParams(use_tc_tiling_on_sc=True),
    )
    return kern(x_num_flat, x_cat_flat, w_flat, bias_flat, tables_flat, cls_flat)


def kernel(x_num, x_cat, W_num, b_num, tables, cls_token):
    x_num_flat = jnp.pad(x_num, ((0, 0), (0, 16 - _NN))).reshape(_B * 16)
    x_cat_flat = x_cat.astype(jnp.int32).reshape(_B * _NC)
    tables_flat = tables.reshape(_NC * _V, _D)
    w_flat = W_num.reshape(_NN * _D)
    bias_flat = b_num.reshape(_NN * _D)
    cls_flat = cls_token.reshape(_D)
    out2 = _tokenize(x_num_flat, x_cat_flat, w_flat, bias_flat,
                     tables_flat, cls_flat)
    return out2.reshape(_B, _T, _D)


# speedup vs baseline: 2.7239x; 1.0227x over previous
import jax
import jax.numpy as jnp
from jax import lax
from jax.experimental import pallas as pl
from jax.experimental.pallas import tpu as pltpu
from jax.experimental.pallas import tpu_sc as plsc

_B = 4096
_NN = 13
_NC = 26
_V = 100000
_D = 64
_T = 1 + _NN + _NC

_NW = 32
_BPW = _B // _NW
_CB = 8
_NCHUNK = _BPW // _CB

_NCAT = _CB * _NC


def _sc_body(xnum_hbm, xcat_hbm, w_hbm, bias_hbm, tab_hbm, cls_hbm, out_hbm,
             idx_v, asm0_v, asm1_v, xnum_v, w_v, bias_v, cls_v,
             gsem0, gsem1, osem0, osem1):
    cid = lax.axis_index("c")
    sid = lax.axis_index("s")
    wid = sid * 2 + cid
    base = wid * _BPW

    pltpu.sync_copy(w_hbm, w_v)
    pltpu.sync_copy(bias_hbm, bias_v)
    pltpu.sync_copy(cls_hbm, cls_v)
    pltpu.sync_copy(xcat_hbm.at[pl.ds(base * _NC, _BPW * _NC)], idx_v)
    pltpu.sync_copy(xnum_hbm.at[pl.ds(base * 16, _BPW * 16)], xnum_v)

    def _fire(c, asm_v, gsem, osem, wait_prev):
        b0 = base + c * _CB
        dst = out_hbm.at[pl.ds(b0 * _T, _CB * _T)]

        @pl.when(wait_prev)
        def _():
            pltpu.make_async_copy(asm_v, dst, osem).wait()

        o = c * _NCAT
        for j in range(_NCAT // 16):
            vv = idx_v[pl.ds(o + j * 16, 16)]
            for i in range(16):
                p = j * 16 + i
                b, f = divmod(p, _NC)
                pltpu.async_copy(
                    tab_hbm.at[vv[i] + f * _V],
                    asm_v.at[b * _T + 1 + _NN + f], gsem)
        return dst

    def _finish(c, asm_v, gsem, osem, dst):
        for b in range(_CB):
            xv = xnum_v[pl.ds((c * _CB + b) * 16, 16)]
            for k in range(_D // 16):
                asm_v[b * _T, pl.ds(k * 16, 16)] = cls_v[pl.ds(k * 16, 16)]
            for f in range(_NN):
                xs = xv[f]
                for k in range(_D // 16):
                    s = pl.ds(k * 16, 16)
                    asm_v[b * _T + 1 + f, s] = (
                        xs * w_v[pl.ds(f * _D + k * 16, 16)]
                        + bias_v[pl.ds(f * _D + k * 16, 16)])

        pltpu.make_async_copy(
            tab_hbm.at[pl.ds(0, _NCAT)], asm_v.at[pl.ds(0, _NCAT)], gsem
        ).wait()

        pltpu.async_copy(asm_v, dst, osem)

    def _pair(p, _):
        dst0 = _fire(2 * p, asm0_v, gsem0, osem0, p >= 1)
        dst1 = _fire(2 * p + 1, asm1_v, gsem1, osem1, p >= 1)
        _finish(2 * p, asm0_v, gsem0, osem0, dst0)
        _finish(2 * p + 1, asm1_v, gsem1, osem1, dst1)
        return 0

    lax.fori_loop(0, _NCHUNK // 2, _pair, 0)

    last = out_hbm.at[pl.ds(base * _T, _CB * _T)]
    pltpu.make_async_copy(asm0_v, last, osem0).wait()
    pltpu.make_async_copy(asm1_v, last, osem1).wait()


@jax.jit
def _tokenize(x_num_flat, x_cat_flat, w_flat, bias_flat, tables_flat, cls_flat):
    mesh = plsc.VectorSubcoreMesh(core_axis_name="c", subcore_axis_name="s")
    kern = pl.kernel(
        _sc_body,
        out_type=jax.ShapeDtypeStruct((_B * _T, _D), jnp.float32),
        mesh=mesh,
        scratch_types=[
            pltpu.VMEM((_BPW * _NC,), jnp.int32),
            pltpu.VMEM((_CB * _T, _D), jnp.float32),
            pltpu.VMEM((_CB * _T, _D), jnp.float32),
            pltpu.VMEM((_BPW * 16,), jnp.float32),
            pltpu.VMEM((_NN * _D,), jnp.float32),
            pltpu.VMEM((_NN * _D,), jnp.float32),
            pltpu.VMEM((_D,), jnp.float32),
            pltpu.SemaphoreType.DMA,
            pltpu.SemaphoreType.DMA,
            pltpu.SemaphoreType.DMA,
            pltpu.SemaphoreType.DMA,
        ],
        compiler_params=pltpu.CompilerParams(use_tc_tiling_on_sc=True),
    )
    return kern(x_num_flat, x_cat_flat, w_flat, bias_flat, tables_flat, cls_flat)


def kernel(x_num, x_cat, W_num, b_num, tables, cls_token):
    x_num_flat = jnp.pad(x_num, ((0, 0), (0, 16 - _NN))).reshape(_B * 16)
    x_cat_flat = x_cat.astype(jnp.int32).reshape(_B * _NC)
    tables_flat = tables.reshape(_NC * _V, _D)
    w_flat = W_num.reshape(_NN * _D)
    bias_flat = b_num.reshape(_NN * _D)
    cls_flat = cls_token.reshape(_D)
    out2 = _tokenize(x_num_flat, x_cat_flat, w_flat, bias_flat,
                     tables_flat, cls_flat)
    return out2.reshape(_B, _T, _D)


# speedup vs baseline: 2.7549x; 1.0114x over previous
import jax
import jax.numpy as jnp
from jax import lax
from jax.experimental import pallas as pl
from jax.experimental.pallas import tpu as pltpu
from jax.experimental.pallas import tpu_sc as plsc

_B = 4096
_NN = 13
_NC = 26
_V = 100000
_D = 64
_T = 1 + _NN + _NC

_NW = 32
_BPW = _B // _NW
_CB = 8
_NCHUNK = _BPW // _CB

_NCAT = _CB * _NC


def _sc_body(xnum_hbm, xcat_hbm, w_hbm, bias_hbm, tab_hbm, cls_hbm, out_hbm,
             idx_v, asm0_v, asm1_v, xnum_v, w_v, bias_v, cls_v,
             gsem0, gsem1, osem0, osem1):
    cid = lax.axis_index("c")
    sid = lax.axis_index("s")
    wid = sid * 2 + cid
    base = wid * _BPW

    pltpu.sync_copy(w_hbm, w_v)
    pltpu.sync_copy(bias_hbm, bias_v)
    pltpu.sync_copy(cls_hbm, cls_v)
    pltpu.sync_copy(xcat_hbm.at[pl.ds(base * _NC, _BPW * _NC)], idx_v)
    pltpu.sync_copy(xnum_hbm.at[pl.ds(base * 16, _BPW * 16)], xnum_v)

    def _fire(c, asm_v, gsem, osem, wait_prev):
        b0 = base + c * _CB
        dst = out_hbm.at[pl.ds(b0 * _T, _CB * _T)]

        @pl.when(wait_prev)
        def _():
            pltpu.make_async_copy(asm_v, dst, osem).wait()

        o = c * _NCAT
        for j in range(_NCAT // 16):
            vv = idx_v[pl.ds(o + j * 16, 16)]
            for i in range(16):
                p = j * 16 + i
                b, f = divmod(p, _NC)
                pltpu.async_copy(
                    tab_hbm.at[vv[i] + f * _V],
                    asm_v.at[b * _T + 1 + _NN + f], gsem)
        return dst

    def _finish(c, asm_v, gsem, osem, dst):
        clsk = [cls_v[pl.ds(k * 16, 16)] for k in range(_D // 16)]
        xvs = [xnum_v[pl.ds((c * _CB + b) * 16, 16)] for b in range(_CB)]
        for b in range(_CB):
            for k in range(_D // 16):
                asm_v[b * _T, pl.ds(k * 16, 16)] = clsk[k]
        for f in range(_NN):
            wk = [w_v[pl.ds(f * _D + k * 16, 16)] for k in range(_D // 16)]
            bk = [bias_v[pl.ds(f * _D + k * 16, 16)] for k in range(_D // 16)]
            for b in range(_CB):
                xs = xvs[b][f]
                for k in range(_D // 16):
                    s = pl.ds(k * 16, 16)
                    asm_v[b * _T + 1 + f, s] = xs * wk[k] + bk[k]

        pltpu.make_async_copy(
            tab_hbm.at[pl.ds(0, _NCAT)], asm_v.at[pl.ds(0, _NCAT)], gsem
        ).wait()

        pltpu.async_copy(asm_v, dst, osem)

    def _pair(p, _):
        dst0 = _fire(2 * p, asm0_v, gsem0, osem0, p >= 1)
        dst1 = _fire(2 * p + 1, asm1_v, gsem1, osem1, p >= 1)
        _finish(2 * p, asm0_v, gsem0, osem0, dst0)
        _finish(2 * p + 1, asm1_v, gsem1, osem1, dst1)
        return 0

    lax.fori_loop(0, _NCHUNK // 2, _pair, 0)

    last = out_hbm.at[pl.ds(base * _T, _CB * _T)]
    pltpu.make_async_copy(asm0_v, last, osem0).wait()
    pltpu.make_async_copy(asm1_v, last, osem1).wait()


@jax.jit
def _tokenize(x_num_flat, x_cat_flat, w_flat, bias_flat, tables_flat, cls_flat):
    mesh = plsc.VectorSubcoreMesh(core_axis_name="c", subcore_axis_name="s")
    kern = pl.kernel(
        _sc_body,
        out_type=jax.ShapeDtypeStruct((_B * _T, _D), jnp.float32),
        mesh=mesh,
        scratch_types=[
            pltpu.VMEM((_BPW * _NC,), jnp.int32),
            pltpu.VMEM((_CB * _T, _D), jnp.float32),
            pltpu.VMEM((_CB * _T, _D), jnp.float32),
            pltpu.VMEM((_BPW * 16,), jnp.float32),
            pltpu.VMEM((_NN * _D,), jnp.float32),
            pltpu.VMEM((_NN * _D,), jnp.float32),
            pltpu.VMEM((_D,), jnp.float32),
            pltpu.SemaphoreType.DMA,
            pltpu.SemaphoreType.DMA,
            pltpu.SemaphoreType.DMA,
            pltpu.SemaphoreType.DMA,
        ],
        compiler_params=pltpu.CompilerParams(use_tc_tiling_on_sc=True),
    )
    return kern(x_num_flat, x_cat_flat, w_flat, bias_flat, tables_flat, cls_flat)


def kernel(x_num, x_cat, W_num, b_num, tables, cls_token):
    x_num_flat = jnp.pad(x_num, ((0, 0), (0, 16 - _NN))).reshape(_B * 16)
    x_cat_flat = x_cat.astype(jnp.int32).reshape(_B * _NC)
    tables_flat = tables.reshape(_NC * _V, _D)
    w_flat = W_num.reshape(_NN * _D)
    bias_flat = b_num.reshape(_NN * _D)
    cls_flat = cls_token.reshape(_D)
    out2 = _tokenize(x_num_flat, x_cat_flat, w_flat, bias_flat,
                     tables_flat, cls_flat)
    return out2.reshape(_B, _T, _D)
